# Initial kernel scaffold; baseline (speedup 1.0000x reference)
import jax
import jax.numpy as jnp
from jax.experimental import pallas as pl


def kernel(predictions):
    raise NotImplementedError("write your pallas kernel here")


# speedup vs baseline: 3.3192x; 3.3192x over previous
import functools

import jax
import jax.numpy as jnp
import numpy as np
from jax import lax
from jax.experimental import pallas as pl

_STEPS = [32, 16, 8, 4, 2, 1]
_NUM_CLASSES = 20
_BATCH = 8
_N_ANCH = 9 * sum(n * n for n in _STEPS)
_N_PAD = 12288
_IOU_THRESH = 0.5
_SCORE_THRESH = 0.03
_MAX_PER_CLASS = 100
_MAX_TOTAL = 100
_NEG = -1e9
_BIGI = 2**30


def _anchors_t() -> np.ndarray:
    all_b = []
    scales = [2.0 ** x for x in [0.0, 1.0 / 3.0, 2.0 / 3.0]]
    ratios = [0.5, 1.0, 2.0]
    for n in _STEPS:
        fw = 1.0 / n
        rows, cols = np.meshgrid(np.arange(n), np.arange(n), indexing='ij')
        cx = (cols + 0.5) * fw
        cy = (rows + 0.5) * fw
        whs = np.array(
            [[s * np.sqrt(r) * fw, s / np.sqrt(r) * fw]
             for s in scales for r in ratios], dtype=np.float64)
        cxcy = np.stack([cx, cy], axis=-1).reshape(n * n, 1, 2)
        loc = np.broadcast_to(cxcy, (n * n, 9, 2))
        whb = np.broadcast_to(whs.reshape(1, 9, 2), (n * n, 9, 2))
        all_b.append(np.concatenate([loc, whb], axis=-1).reshape(-1, 4))
    anch = np.concatenate(all_b, 0).astype(np.float32)
    pad = np.zeros((_N_PAD - _N_ANCH, 4), dtype=np.float32)
    return np.concatenate([anch, pad], 0).T.copy()


def _nms_kernel(sc_ref, bp_ref, anc_ref, out_ref):
    C = _NUM_CLASSES
    s = sc_ref[0]
    bp = bp_ref[0]
    anc = anc_ref[...]

    acx, acy, aw, ah = anc[0:1], anc[1:2], anc[2:3], anc[3:4]
    bcx = bp[0:1] * 0.1
    bcy = bp[1:2] * 0.1
    bw = bp[2:3] * 0.2
    bh = bp[3:4] * 0.2
    cx = bcx * aw + acx
    cy = bcy * ah + acy
    w = jnp.exp(bw) * aw
    h = jnp.exp(bh) * ah
    x1 = cx - w * 0.5
    y1 = cy - h * 0.5
    x2 = cx + w * 0.5
    y2 = cy + h * 0.5
    areas = (x2 - x1) * (y2 - y1)

    lane_iota = lax.broadcasted_iota(jnp.int32, (C, _N_PAD), 1)
    pick_iota = lax.broadcasted_iota(jnp.int32, (C, 128), 1)

    cur0 = jnp.where(s > _SCORE_THRESH, s, _NEG)

    zrow = jnp.zeros((C, 128), jnp.float32)

    def nms_step(t, carry):
        cur, sx1, sy1, sx2, sy2, ssc, svd = carry
        bs = jnp.max(cur, axis=1, keepdims=True)
        eq = cur == bs
        fidx = jnp.min(jnp.where(eq, lane_iota, _BIGI), axis=1,
                       keepdims=True)
        oh = lane_iota == fidx
        ninf = jnp.float32(-3e38)
        bb0 = jnp.max(jnp.where(oh, x1, ninf), axis=1, keepdims=True)
        bb1 = jnp.max(jnp.where(oh, y1, ninf), axis=1, keepdims=True)
        bb2 = jnp.max(jnp.where(oh, x2, ninf), axis=1, keepdims=True)
        bb3 = jnp.max(jnp.where(oh, y2, ninf), axis=1, keepdims=True)
        barea = (bb2 - bb0) * (bb3 - bb1)
        ix1 = jnp.maximum(bb0, x1)
        iy1 = jnp.maximum(bb1, y1)
        ix2 = jnp.minimum(bb2, x2)
        iy2 = jnp.minimum(bb3, y2)
        inter = jnp.maximum(ix2 - ix1, 0.0) * jnp.maximum(iy2 - iy1, 0.0)
        iou = inter / (areas + barea - inter + 1e-8)
        valid = bs > _NEG / 2.0
        supp = (iou >= _IOU_THRESH) | oh
        newc = jnp.where(valid & supp, _NEG, cur)
        sel = pick_iota == t
        sx1 = jnp.where(sel, jnp.where(valid, bb0, 0.0), sx1)
        sy1 = jnp.where(sel, jnp.where(valid, bb1, 0.0), sy1)
        sx2 = jnp.where(sel, jnp.where(valid, bb2, 0.0), sx2)
        sy2 = jnp.where(sel, jnp.where(valid, bb3, 0.0), sy2)
        ssc = jnp.where(sel, jnp.where(valid, bs, 0.0), ssc)
        svd = jnp.where(sel, jnp.where(valid, 1.0, 0.0), svd)
        return newc, sx1, sy1, sx2, sy2, ssc, svd

    _, sx1, sy1, sx2, sy2, ssc, svd = lax.fori_loop(
        0, _MAX_PER_CLASS, nms_step,
        (cur0, zrow, zrow, zrow, zrow, zrow, zrow))

    key = jnp.where(svd > 0.5, ssc, _NEG)
    c_iota = lax.broadcasted_iota(jnp.int32, (C, 128), 0)
    fmat = c_iota * 128 + pick_iota
    out_iota = lax.broadcasted_iota(jnp.int32, (1, 128), 1)
    zo = jnp.zeros((1, 128), jnp.float32)

    def merge_step(t2, carry):
        key, ob0, ob1, ob2, ob3, osc, ocl, nv = carry
        mv = jnp.max(key)
        eq2 = key == mv
        bf = jnp.min(jnp.where(eq2, fmat, _BIGI))
        oh2 = fmat == bf
        ninf = jnp.float32(-3e38)
        g0 = jnp.max(jnp.where(oh2, sx1, ninf))
        g1 = jnp.max(jnp.where(oh2, sy1, ninf))
        g2 = jnp.max(jnp.where(oh2, sx2, ninf))
        g3 = jnp.max(jnp.where(oh2, sy2, ninf))
        vflag = mv > _NEG / 2.0
        clsf = (bf // 128).astype(jnp.float32)
        vz = jnp.float32(0.0)
        c0 = jnp.where(vflag, jnp.clip(g0, 0.0, 1.0), vz)
        c1 = jnp.where(vflag, jnp.clip(g1, 0.0, 1.0), vz)
        c2 = jnp.where(vflag, jnp.clip(g2, 0.0, 1.0), vz)
        c3 = jnp.where(vflag, jnp.clip(g3, 0.0, 1.0), vz)
        cs = jnp.where(vflag, mv, vz)
        cc = jnp.where(vflag, clsf, vz)
        sel2 = out_iota == t2
        ob0 = jnp.where(sel2, c0, ob0)
        ob1 = jnp.where(sel2, c1, ob1)
        ob2 = jnp.where(sel2, c2, ob2)
        ob3 = jnp.where(sel2, c3, ob3)
        osc = jnp.where(sel2, cs, osc)
        ocl = jnp.where(sel2, cc, ocl)
        nv = nv + jnp.where(vflag, 1.0, 0.0)
        key = jnp.where(oh2, jnp.float32(-3e38), key)
        return key, ob0, ob1, ob2, ob3, osc, ocl, nv

    _, ob0, ob1, ob2, ob3, osc, ocl, nv = lax.fori_loop(
        0, _MAX_TOTAL, merge_step,
        (key, zo, zo, zo, zo, zo, zo, jnp.float32(0.0)))

    out_ref[0, 0:1, :] = ob0
    out_ref[0, 1:2, :] = ob1
    out_ref[0, 2:3, :] = ob2
    out_ref[0, 3:4, :] = ob3
    out_ref[0, 4:5, :] = osc
    out_ref[0, 5:6, :] = ocl
    out_ref[0, 6:7, :] = jnp.full((1, 128), nv, jnp.float32)
    out_ref[0, 7:8, :] = zo


@jax.jit
def _run(predictions, anchors_t):
    B, N, C = _BATCH, _N_ANCH, _NUM_CLASSES
    scores = jnp.transpose(predictions[:, :, 4:], (0, 2, 1))
    box_pred = jnp.transpose(predictions[:, :, :4], (0, 2, 1))
    padn = _N_PAD - N
    scores = jnp.pad(scores, ((0, 0), (0, 0), (0, padn)))
    box_pred = jnp.pad(box_pred, ((0, 0), (0, 0), (0, padn)))

    out = pl.pallas_call(
        _nms_kernel,
        grid=(B,),
        in_specs=[
            pl.BlockSpec((1, C, _N_PAD), lambda i: (i, 0, 0)),
            pl.BlockSpec((1, 4, _N_PAD), lambda i: (i, 0, 0)),
            pl.BlockSpec((4, _N_PAD), lambda i: (0, 0)),
        ],
        out_specs=pl.BlockSpec((1, 8, 128), lambda i: (i, 0, 0)),
        out_shape=jax.ShapeDtypeStruct((B, 8, 128), jnp.float32),
    )(scores, box_pred, anchors_t)

    M = _MAX_TOTAL
    boxes = jnp.transpose(out[:, 0:4, :M], (0, 2, 1))
    nmsed_scores = out[:, 4, :M]
    nmsed_classes = out[:, 5, :M]
    n_valid = out[:, 6, 0].astype(jnp.int32)
    return boxes, nmsed_scores, nmsed_classes, n_valid


_ANCHORS_T = jnp.asarray(_anchors_t())


def kernel(predictions):
    return _run(predictions, _ANCHORS_T)


# speedup vs baseline: 8.7631x; 2.6402x over previous
import functools

import jax
import jax.numpy as jnp
import numpy as np
from jax import lax
from jax.experimental import pallas as pl
from jax.experimental.pallas import tpu as pltpu
from jax.experimental.pallas import tpu_sc as plsc

_STEPS = [32, 16, 8, 4, 2, 1]
_NUM_CLASSES = 20
_BATCH = 8
_N_ANCH = 9 * sum(n * n for n in _STEPS)
_N_PAD = 12288
_NPROB = _BATCH * _NUM_CLASSES
_IOU_THRESH = 0.5
_SCORE_THRESH = 0.03
_MAX_PER_CLASS = 100
_MAX_TOTAL = 100
_NEG = -1e9
_BIGI = 2**30
_KCAP1 = 512
_NV1 = _KCAP1 // 16
_NV2 = _N_PAD // 16
_CBUF = _N_PAD + 16


def _anchors_t() -> np.ndarray:
    all_b = []
    scales = [2.0 ** x for x in [0.0, 1.0 / 3.0, 2.0 / 3.0]]
    ratios = [0.5, 1.0, 2.0]
    for n in _STEPS:
        fw = 1.0 / n
        rows, cols = np.meshgrid(np.arange(n), np.arange(n), indexing='ij')
        cx = (cols + 0.5) * fw
        cy = (rows + 0.5) * fw
        whs = np.array(
            [[s * np.sqrt(r) * fw, s / np.sqrt(r) * fw]
             for s in scales for r in ratios], dtype=np.float64)
        cxcy = np.stack([cx, cy], axis=-1).reshape(n * n, 1, 2)
        loc = np.broadcast_to(cxcy, (n * n, 9, 2))
        whb = np.broadcast_to(whs.reshape(1, 9, 2), (n * n, 9, 2))
        all_b.append(np.concatenate([loc, whb], axis=-1).reshape(-1, 4))
    anch = np.concatenate(all_b, 0).astype(np.float32)
    pad = np.zeros((_N_PAD - _N_ANCH, 4), dtype=np.float32)
    return np.concatenate([anch, pad], 0).T.copy()


def _decode_kernel(bp_ref, anc_ref, out_ref):
    bp = bp_ref[0]
    anc = anc_ref[...]
    acx, acy, aw, ah = anc[0:1], anc[1:2], anc[2:3], anc[3:4]
    cx = bp[0:1] * 0.1 * aw + acx
    cy = bp[1:2] * 0.1 * ah + acy
    w = jnp.exp(bp[2:3] * 0.2) * aw
    h = jnp.exp(bp[3:4] * 0.2) * ah
    x1 = cx - w * 0.5
    y1 = cy - h * 0.5
    x2 = cx + w * 0.5
    y2 = cy + h * 0.5
    ar = (x2 - x1) * (y2 - y1)
    out_ref[0, :, :] = jnp.concatenate([x1, y1, x2, y2, ar], axis=0)


def _sc_nms(sc_ref, t0, t1, t2, t3, t4, out_ref,
            srow, cs, cidx, cx1, cy1, cx2, cy2, car, picks, sem):
    NC = 2
    wid = lax.axis_index("s") * NC + lax.axis_index("c")
    iota16 = lax.iota(jnp.int32, 16)
    negv = jnp.full((16,), _NEG, jnp.float32)

    def count_above(tau):
        def body(j, acc):
            v = srow[pl.ds(j * 16, 16)]
            return acc + (v > tau).astype(jnp.int32)
        acc = lax.fori_loop(0, _NV2, body, jnp.zeros((16,), jnp.int32))
        return jnp.sum(acc)

    def row_max():
        def body(j, m):
            return jnp.maximum(m, srow[pl.ds(j * 16, 16)])
        return jnp.max(lax.fori_loop(0, _NV2, body, negv))

    def prefill(nv, safe_idx):
        pad_i = jnp.full((16,), safe_idx, jnp.int32)
        def body(j, _):
            cs[pl.ds(j * 16, 16)] = negv
            cidx[pl.ds(j * 16, 16)] = pad_i
            return 0
        lax.fori_loop(0, nv + 1, body, 0)

    def compact(tau, base_idx):
        def body(j, cnt):
            v = srow[pl.ds(j * 16, 16)]
            m = v > tau
            plsc.store_compressed(cs.at[pl.ds(cnt, 16)], v, mask=m)
            gi = base_idx + j * 16 + iota16
            plsc.store_compressed(cidx.at[pl.ds(cnt, 16)], gi, mask=m)
            return cnt + jnp.sum(m.astype(jnp.int32))
        return lax.fori_loop(0, _NV2, body, 0)

    def gather_boxes(nchunks):
        for tbl, dst in ((t0, cx1), (t1, cy1), (t2, cx2), (t3, cy2),
                         (t4, car)):
            def body(q, _, tbl=tbl, dst=dst):
                idx = cidx.at[pl.ds(q * 128, 128)]
                pltpu.async_copy(tbl.at[idx], dst.at[pl.ds(q * 128, 128)],
                                 sem).wait()
                return 0
            lax.fori_loop(0, nchunks, body, 0)

    def greedy(nv):
        def w_cond(c):
            t, alive = c
            return (t < _MAX_PER_CLASS) & alive

        def w_body(c):
            t, _ = c

            def am(j, carry):
                bm, bj = carry
                v = cs[pl.ds(j * 16, 16)]
                better = v > bm
                return (jnp.where(better, v, bm),
                        jnp.where(better, j, bj))
            bm, bj = lax.fori_loop(0, nv, am,
                                   (negv, jnp.zeros((16,), jnp.int32)))
            bs = jnp.max(bm)
            alive = bs > _NEG / 2.0
            gi = jnp.where(bm == bs, bj * 16 + iota16, _BIGI)
            pos = jnp.min(gi)

            @pl.when(alive)
            def _():
                b0 = cx1[pl.ds(pos, 16)][0]
                b1 = cy1[pl.ds(pos, 16)][0]
                b2 = cx2[pl.ds(pos, 16)][0]
                b3 = cy2[pl.ds(pos, 16)][0]
                ba = car[pl.ds(pos, 16)][0]

                def su(j, _):
                    v = cs[pl.ds(j * 16, 16)]
                    vx1 = cx1[pl.ds(j * 16, 16)]
                    vy1 = cy1[pl.ds(j * 16, 16)]
                    vx2 = cx2[pl.ds(j * 16, 16)]
                    vy2 = cy2[pl.ds(j * 16, 16)]
                    va = car[pl.ds(j * 16, 16)]
                    ix1 = jnp.maximum(b0, vx1)
                    iy1 = jnp.maximum(b1, vy1)
                    ix2 = jnp.minimum(b2, vx2)
                    iy2 = jnp.minimum(b3, vy2)
                    inter = (jnp.maximum(ix2 - ix1, 0.0)
                             * jnp.maximum(iy2 - iy1, 0.0))
                    iou = inter / (va + ba - inter + 1e-8)
                    supp = (iou >= _IOU_THRESH) | (j * 16 + iota16 == pos)
                    cs[pl.ds(j * 16, 16)] = jnp.where(supp, _NEG, v)
                    return 0
                lax.fori_loop(0, nv, su, 0)
                colv = jnp.full((16,), t, jnp.int32)
                lane0 = iota16 == 0
                for r, val in enumerate((b0, b1, b2, b3, bs,
                                         jnp.float32(1.0))):
                    plsc.store_scatter(
                        picks, [jnp.full((16,), r, jnp.int32), colv],
                        jnp.full((16,), val, jnp.float32), mask=lane0)

            return t + alive.astype(jnp.int32), alive

        t, _ = lax.while_loop(w_cond, w_body, (0, True))
        return t

    def problem(k, _):
        p = wid * 5 + k
        img = p // _NUM_CLASSES
        base_idx = img * _N_PAD
        pltpu.sync_copy(sc_ref.at[p], srow)
        for r in range(6):
            for q in range(8):
                picks[r, pl.ds(q * 16, 16)] = jnp.zeros((16,), jnp.float32)

        c03 = count_above(jnp.float32(_SCORE_THRESH))

        def solve_dense():
            prefill(_NV2, base_idx)
            compact(jnp.float32(_SCORE_THRESH), base_idx)
            gather_boxes(_N_PAD // 128)
            greedy(_NV2)

        def solve_small():
            prefill(_NV1, base_idx)
            compact(jnp.float32(_SCORE_THRESH), base_idx)
            gather_boxes(_KCAP1 // 128)
            greedy(_NV1)

        def solve_truncated():
            smax = row_max()

            def b_cond(c):
                lo, hi, tau, cnt, it = c
                return ((cnt < _MAX_PER_CLASS) | (cnt > _KCAP1)) & (it < 24)

            def b_body(c):
                lo, hi, tau, cnt, it = c
                mid = 0.5 * (lo + hi)
                first = jnp.float32(1.80)
                mid = jnp.where((it == 0) & (first > lo) & (first < hi),
                                first, mid)
                cc = count_above(mid)
                lo = jnp.where(cc > _KCAP1, mid, lo)
                hi = jnp.where(cc > _KCAP1, hi, mid)
                return lo, hi, mid, cc, it + 1

            lo0 = jnp.float32(_SCORE_THRESH)
            _, _, tau, cnt, _ = lax.while_loop(
                b_cond, b_body, (lo0, smax, lo0, c03, 0))

            def trunc_path():
                prefill(_NV1, base_idx)
                compact(tau, base_idx)
                gather_boxes(_KCAP1 // 128)
                npicks = greedy(_NV1)
                pl.when(npicks < _MAX_PER_CLASS)(solve_dense)

            lax.cond(cnt > _KCAP1, solve_dense, trunc_path)

        lax.cond(c03 <= _KCAP1, solve_small, solve_truncated)
        pltpu.sync_copy(picks, out_ref.at[p])
        return 0

    lax.fori_loop(0, _NPROB // 32, problem, 0)


def _merge_kernel(x1_ref, y1_ref, x2_ref, y2_ref, sc_ref, vd_ref, out_ref):
    C = _NUM_CLASSES
    sx1, sy1 = x1_ref[0], y1_ref[0]
    sx2, sy2 = x2_ref[0], y2_ref[0]
    ssc, svd = sc_ref[0], vd_ref[0]

    key = jnp.where(svd > 0.5, ssc, _NEG)
    pick_iota = lax.broadcasted_iota(jnp.int32, (C, 128), 1)
    c_iota = lax.broadcasted_iota(jnp.int32, (C, 128), 0)
    fmat = c_iota * 128 + pick_iota
    out_iota = lax.broadcasted_iota(jnp.int32, (1, 128), 1)
    zo = jnp.zeros((1, 128), jnp.float32)

    def merge_step(t2, carry):
        key, ob0, ob1, ob2, ob3, osc, ocl, nv = carry
        mv = jnp.max(key)
        eq2 = key == mv
        bf = jnp.min(jnp.where(eq2, fmat, _BIGI))
        oh2 = fmat == bf
        ninf = jnp.float32(-3e38)
        g0 = jnp.max(jnp.where(oh2, sx1, ninf))
        g1 = jnp.max(jnp.where(oh2, sy1, ninf))
        g2 = jnp.max(jnp.where(oh2, sx2, ninf))
        g3 = jnp.max(jnp.where(oh2, sy2, ninf))
        vflag = mv > _NEG / 2.0
        clsf = (bf // 128).astype(jnp.float32)
        vz = jnp.float32(0.0)
        c0 = jnp.where(vflag, jnp.clip(g0, 0.0, 1.0), vz)
        c1 = jnp.where(vflag, jnp.clip(g1, 0.0, 1.0), vz)
        c2 = jnp.where(vflag, jnp.clip(g2, 0.0, 1.0), vz)
        c3 = jnp.where(vflag, jnp.clip(g3, 0.0, 1.0), vz)
        cs = jnp.where(vflag, mv, vz)
        cc = jnp.where(vflag, clsf, vz)
        sel2 = out_iota == t2
        ob0 = jnp.where(sel2, c0, ob0)
        ob1 = jnp.where(sel2, c1, ob1)
        ob2 = jnp.where(sel2, c2, ob2)
        ob3 = jnp.where(sel2, c3, ob3)
        osc = jnp.where(sel2, cs, osc)
        ocl = jnp.where(sel2, cc, ocl)
        nv = nv + jnp.where(vflag, 1.0, 0.0)
        key = jnp.where(oh2, jnp.float32(-3e38), key)
        return key, ob0, ob1, ob2, ob3, osc, ocl, nv

    _, ob0, ob1, ob2, ob3, osc, ocl, nv = lax.fori_loop(
        0, _MAX_TOTAL, merge_step,
        (key, zo, zo, zo, zo, zo, zo, jnp.float32(0.0)))

    out_ref[0, 0:1, :] = ob0
    out_ref[0, 1:2, :] = ob1
    out_ref[0, 2:3, :] = ob2
    out_ref[0, 3:4, :] = ob3
    out_ref[0, 4:5, :] = osc
    out_ref[0, 5:6, :] = ocl
    out_ref[0, 6:7, :] = jnp.full((1, 128), nv, jnp.float32)
    out_ref[0, 7:8, :] = zo


@jax.jit
def _run(predictions, anchors_t):
    B, N, C = _BATCH, _N_ANCH, _NUM_CLASSES
    padn = _N_PAD - N
    box_pred = jnp.transpose(predictions[:, :, :4], (0, 2, 1))
    box_pred = jnp.pad(box_pred, ((0, 0), (0, 0), (0, padn)))
    scores = jnp.transpose(predictions[:, :, 4:], (0, 2, 1))
    scores = jnp.pad(scores, ((0, 0), (0, 0), (0, padn)))
    scores = scores.reshape(_NPROB, _N_PAD)

    table = pl.pallas_call(
        _decode_kernel,
        grid=(B,),
        in_specs=[
            pl.BlockSpec((1, 4, _N_PAD), lambda i: (i, 0, 0)),
            pl.BlockSpec((4, _N_PAD), lambda i: (0, 0)),
        ],
        out_specs=pl.BlockSpec((1, 5, _N_PAD), lambda i: (i, 0, 0)),
        out_shape=jax.ShapeDtypeStruct((B, 5, _N_PAD), jnp.float32),
    )(box_pred, anchors_t)
    tflat = jnp.transpose(table, (1, 0, 2)).reshape(5, B * _N_PAD)

    mesh = plsc.VectorSubcoreMesh(core_axis_name="c", subcore_axis_name="s",
                                  num_cores=2, num_subcores=16)
    picks = pl.kernel(
        _sc_nms,
        out_type=jax.ShapeDtypeStruct((_NPROB, 6, 128), jnp.float32),
        mesh=mesh,
        compiler_params=pltpu.CompilerParams(needs_layout_passes=False),
        scratch_types=[
            pltpu.VMEM((_N_PAD,), jnp.float32),
            pltpu.VMEM((_CBUF,), jnp.float32),
            pltpu.VMEM((_CBUF,), jnp.int32),
            pltpu.VMEM((_CBUF,), jnp.float32),
            pltpu.VMEM((_CBUF,), jnp.float32),
            pltpu.VMEM((_CBUF,), jnp.float32),
            pltpu.VMEM((_CBUF,), jnp.float32),
            pltpu.VMEM((_CBUF,), jnp.float32),
            pltpu.VMEM((6, 128), jnp.float32),
            pltpu.SemaphoreType.DMA,
        ],
    )(scores, tflat[0], tflat[1], tflat[2], tflat[3], tflat[4])

    pt = jnp.transpose(picks, (1, 0, 2)).reshape(6, B, C, 128)

    out = pl.pallas_call(
        _merge_kernel,
        grid=(B,),
        in_specs=[pl.BlockSpec((1, C, 128), lambda i: (i, 0, 0))] * 6,
        out_specs=pl.BlockSpec((1, 8, 128), lambda i: (i, 0, 0)),
        out_shape=jax.ShapeDtypeStruct((B, 8, 128), jnp.float32),
    )(pt[0], pt[1], pt[2], pt[3], pt[4], pt[5])

    M = _MAX_TOTAL
    boxes = jnp.transpose(out[:, 0:4, :M], (0, 2, 1))
    nmsed_scores = out[:, 4, :M]
    nmsed_classes = out[:, 5, :M]
    n_valid = out[:, 6, 0].astype(jnp.int32)
    return boxes, nmsed_scores, nmsed_classes, n_valid


_ANCHORS_T = _anchors_t()


def kernel(predictions):
    return _run(predictions, _ANCHORS_T)


# speedup vs baseline: 21.2316x; 2.4228x over previous
import functools

import jax
import jax.numpy as jnp
import numpy as np
from jax import lax
from jax.experimental import pallas as pl
from jax.experimental.pallas import tpu as pltpu
from jax.experimental.pallas import tpu_sc as plsc

_STEPS = [32, 16, 8, 4, 2, 1]
_NUM_CLASSES = 20
_BATCH = 8
_N_ANCH = 9 * sum(n * n for n in _STEPS)
_N_PAD = 12288
_NPROB = _BATCH * _NUM_CLASSES
_IOU_THRESH = 0.5
_SCORE_THRESH = 0.03
_MAX_PER_CLASS = 100
_MAX_TOTAL = 100
_NEG = -1e9
_BIGI = 2**30
_KCAP1 = 256
_NV1 = _KCAP1 // 16
_NV2 = _N_PAD // 16
_CBUF = _N_PAD + 16


def _anchors_t() -> np.ndarray:
    all_b = []
    scales = [2.0 ** x for x in [0.0, 1.0 / 3.0, 2.0 / 3.0]]
    ratios = [0.5, 1.0, 2.0]
    for n in _STEPS:
        fw = 1.0 / n
        rows, cols = np.meshgrid(np.arange(n), np.arange(n), indexing='ij')
        cx = (cols + 0.5) * fw
        cy = (rows + 0.5) * fw
        whs = np.array(
            [[s * np.sqrt(r) * fw, s / np.sqrt(r) * fw]
             for s in scales for r in ratios], dtype=np.float64)
        cxcy = np.stack([cx, cy], axis=-1).reshape(n * n, 1, 2)
        loc = np.broadcast_to(cxcy, (n * n, 9, 2))
        whb = np.broadcast_to(whs.reshape(1, 9, 2), (n * n, 9, 2))
        all_b.append(np.concatenate([loc, whb], axis=-1).reshape(-1, 4))
    anch = np.concatenate(all_b, 0).astype(np.float32)
    pad = np.zeros((_N_PAD - _N_ANCH, 4), dtype=np.float32)
    return np.concatenate([anch, pad], 0).T.copy()


def _decode_kernel(bp_ref, anc_ref, out_ref):
    bp = bp_ref[0]
    anc = anc_ref[...]
    acx, acy, aw, ah = anc[0:1], anc[1:2], anc[2:3], anc[3:4]
    cx = bp[0:1] * 0.1 * aw + acx
    cy = bp[1:2] * 0.1 * ah + acy
    w = jnp.exp(bp[2:3] * 0.2) * aw
    h = jnp.exp(bp[3:4] * 0.2) * ah
    x1 = cx - w * 0.5
    y1 = cy - h * 0.5
    x2 = cx + w * 0.5
    y2 = cy + h * 0.5
    ar = (x2 - x1) * (y2 - y1)
    out_ref[0, :, :] = jnp.concatenate([x1, y1, x2, y2, ar], axis=0)


def _sc_nms(sc_ref, t0, t1, t2, t3, t4, out_ref,
            srow, cs, cidx, cx1, cy1, cx2, cy2, car, picks, sem):
    NC = 2
    wid = lax.axis_index("s") * NC + lax.axis_index("c")
    iota16 = lax.iota(jnp.int32, 16)
    negv = jnp.full((16,), _NEG, jnp.float32)

    def count_above(tau):
        def body(j, acc):
            v = srow[pl.ds(j * 16, 16)]
            return acc + (v > tau).astype(jnp.int32)
        acc = lax.fori_loop(0, _NV2, body, jnp.zeros((16,), jnp.int32))
        return jnp.sum(acc)

    def row_max():
        def body(j, m):
            return jnp.maximum(m, srow[pl.ds(j * 16, 16)])
        return jnp.max(lax.fori_loop(0, _NV2, body, negv))

    def prefill(nv, safe_idx):
        pad_i = jnp.full((16,), safe_idx, jnp.int32)
        def body(j, _):
            cs[pl.ds(j * 16, 16)] = negv
            cidx[pl.ds(j * 16, 16)] = pad_i
            return 0
        lax.fori_loop(0, nv + 1, body, 0)

    def compact(tau, base_idx):
        def body(j, cnt):
            v = srow[pl.ds(j * 16, 16)]
            m = v > tau
            plsc.store_compressed(cs.at[pl.ds(cnt, 16)], v, mask=m)
            gi = base_idx + j * 16 + iota16
            plsc.store_compressed(cidx.at[pl.ds(cnt, 16)], gi, mask=m)
            return cnt + jnp.sum(m.astype(jnp.int32))
        return lax.fori_loop(0, _NV2, body, 0)

    def gather_boxes(nchunks):
        for tbl, dst in ((t0, cx1), (t1, cy1), (t2, cx2), (t3, cy2),
                         (t4, car)):
            def body(q, _, tbl=tbl, dst=dst):
                idx = cidx.at[pl.ds(q * 128, 128)]
                pltpu.async_copy(tbl.at[idx], dst.at[pl.ds(q * 128, 128)],
                                 sem).wait()
                return 0
            lax.fori_loop(0, nchunks, body, 0)

    def greedy(nv):
        zeros16i = jnp.zeros((16,), jnp.int32)

        def am(j, carry):
            bm, bj = carry
            v = cs[pl.ds(j * 16, 16)]
            better = v > bm
            return (jnp.where(better, v, bm),
                    jnp.where(better, j, bj))
        bm0, bj0 = lax.fori_loop(0, nv, am, (negv, zeros16i))

        def w_cond(c):
            t, alive, _, _ = c
            return (t < _MAX_PER_CLASS) & alive

        def w_body(c):
            t, _, bm, bj = c
            bs = jnp.max(bm)
            alive = bs > _NEG / 2.0
            gi = jnp.where(bm == bs, bj * 16 + iota16, _BIGI)
            pos = jnp.min(gi)

            def do_pick():
                b0 = cx1[pl.ds(pos, 16)][0]
                b1 = cy1[pl.ds(pos, 16)][0]
                b2 = cx2[pl.ds(pos, 16)][0]
                b3 = cy2[pl.ds(pos, 16)][0]
                ba = car[pl.ds(pos, 16)][0]

                def su(j, carry):
                    nbm, nbj = carry
                    v = cs[pl.ds(j * 16, 16)]
                    vx1 = cx1[pl.ds(j * 16, 16)]
                    vy1 = cy1[pl.ds(j * 16, 16)]
                    vx2 = cx2[pl.ds(j * 16, 16)]
                    vy2 = cy2[pl.ds(j * 16, 16)]
                    va = car[pl.ds(j * 16, 16)]
                    ix1 = jnp.maximum(b0, vx1)
                    iy1 = jnp.maximum(b1, vy1)
                    ix2 = jnp.minimum(b2, vx2)
                    iy2 = jnp.minimum(b3, vy2)
                    inter = (jnp.maximum(ix2 - ix1, 0.0)
                             * jnp.maximum(iy2 - iy1, 0.0))
                    iou = inter / (va + ba - inter + 1e-8)
                    supp = (iou >= _IOU_THRESH) | (j * 16 + iota16 == pos)
                    newv = jnp.where(supp, _NEG, v)
                    cs[pl.ds(j * 16, 16)] = newv
                    better = newv > nbm
                    return (jnp.where(better, newv, nbm),
                            jnp.where(better, j, nbj))
                nbm, nbj = lax.fori_loop(0, nv, su, (negv, zeros16i))
                colv = jnp.full((16,), t, jnp.int32)
                lane0 = iota16 == 0
                for r, val in enumerate((b0, b1, b2, b3, bs,
                                         jnp.float32(1.0))):
                    plsc.store_scatter(
                        picks, [jnp.full((16,), r, jnp.int32), colv],
                        jnp.full((16,), val, jnp.float32), mask=lane0)
                return nbm, nbj

            def no_pick():
                return bm, bj

            nbm, nbj = lax.cond(alive, do_pick, no_pick)
            return t + alive.astype(jnp.int32), alive, nbm, nbj

        t, _, _, _ = lax.while_loop(w_cond, w_body, (0, True, bm0, bj0))
        return t

    def problem(k, _):
        p = wid * 5 + k
        img = p // _NUM_CLASSES
        base_idx = img * _N_PAD
        pltpu.sync_copy(sc_ref.at[p], srow)
        for r in range(6):
            for q in range(8):
                picks[r, pl.ds(q * 16, 16)] = jnp.zeros((16,), jnp.float32)

        c03 = count_above(jnp.float32(_SCORE_THRESH))

        def solve_dense():
            prefill(_NV2, base_idx)
            compact(jnp.float32(_SCORE_THRESH), base_idx)
            gather_boxes(_N_PAD // 128)
            greedy(_NV2)

        def solve_small():
            prefill(_NV1, base_idx)
            compact(jnp.float32(_SCORE_THRESH), base_idx)
            gather_boxes(_KCAP1 // 128)
            greedy(_NV1)

        def solve_truncated():
            smax = row_max()

            def b_cond(c):
                lo, hi, tau, cnt, it = c
                return ((cnt < _MAX_PER_CLASS) | (cnt > _KCAP1)) & (it < 24)

            def b_body(c):
                lo, hi, tau, cnt, it = c
                mid = 0.5 * (lo + hi)
                first = jnp.float32(2.20)
                mid = jnp.where((it == 0) & (first > lo) & (first < hi),
                                first, mid)
                cc = count_above(mid)
                lo = jnp.where(cc > _KCAP1, mid, lo)
                hi = jnp.where(cc > _KCAP1, hi, mid)
                return lo, hi, mid, cc, it + 1

            lo0 = jnp.float32(_SCORE_THRESH)
            _, _, tau, cnt, _ = lax.while_loop(
                b_cond, b_body, (lo0, smax, lo0, c03, 0))

            def trunc_path():
                prefill(_NV1, base_idx)
                compact(tau, base_idx)
                gather_boxes(_KCAP1 // 128)
                npicks = greedy(_NV1)
                pl.when(npicks < _MAX_PER_CLASS)(solve_dense)

            lax.cond(cnt > _KCAP1, solve_dense, trunc_path)

        lax.cond(c03 <= _KCAP1, solve_small, solve_truncated)
        pltpu.sync_copy(picks, out_ref.at[p])
        return 0

    lax.fori_loop(0, _NPROB // 32, problem, 0)


def _merge_kernel(x1_ref, y1_ref, x2_ref, y2_ref, sc_ref, vd_ref, out_ref):
    B, W = _BATCH, _NUM_CLASSES * 128
    sx1, sy1 = x1_ref[...], y1_ref[...]
    sx2, sy2 = x2_ref[...], y2_ref[...]
    ssc, svd = sc_ref[...], vd_ref[...]

    key = jnp.where(svd > 0.5, ssc, _NEG)
    fmat = lax.broadcasted_iota(jnp.int32, (B, W), 1)
    out_iota = lax.broadcasted_iota(jnp.int32, (B, 128), 1)
    zo = jnp.zeros((B, 128), jnp.float32)

    def merge_step(t2, carry):
        key, ob0, ob1, ob2, ob3, osc, ocl, nv = carry
        mv = jnp.max(key, axis=1, keepdims=True)
        eq2 = key == mv
        bf = jnp.min(jnp.where(eq2, fmat, _BIGI), axis=1,
                     keepdims=True)
        oh2 = fmat == bf
        ninf = jnp.float32(-3e38)
        g0 = jnp.max(jnp.where(oh2, sx1, ninf), axis=1, keepdims=True)
        g1 = jnp.max(jnp.where(oh2, sy1, ninf), axis=1, keepdims=True)
        g2 = jnp.max(jnp.where(oh2, sx2, ninf), axis=1, keepdims=True)
        g3 = jnp.max(jnp.where(oh2, sy2, ninf), axis=1, keepdims=True)
        vflag = mv > _NEG / 2.0
        clsf = (bf // 128).astype(jnp.float32)
        vz = jnp.float32(0.0)
        c0 = jnp.where(vflag, jnp.clip(g0, 0.0, 1.0), vz)
        c1 = jnp.where(vflag, jnp.clip(g1, 0.0, 1.0), vz)
        c2 = jnp.where(vflag, jnp.clip(g2, 0.0, 1.0), vz)
        c3 = jnp.where(vflag, jnp.clip(g3, 0.0, 1.0), vz)
        cs = jnp.where(vflag, mv, vz)
        cc = jnp.where(vflag, clsf, vz)
        sel2 = out_iota == t2
        ob0 = jnp.where(sel2, c0, ob0)
        ob1 = jnp.where(sel2, c1, ob1)
        ob2 = jnp.where(sel2, c2, ob2)
        ob3 = jnp.where(sel2, c3, ob3)
        osc = jnp.where(sel2, cs, osc)
        ocl = jnp.where(sel2, cc, ocl)
        nv = nv + jnp.where(vflag, 1.0, 0.0)
        key = jnp.where(oh2, jnp.float32(-3e38), key)
        return key, ob0, ob1, ob2, ob3, osc, ocl, nv

    _, ob0, ob1, ob2, ob3, osc, ocl, nv = lax.fori_loop(
        0, _MAX_TOTAL, merge_step,
        (key, zo, zo, zo, zo, zo, zo, jnp.zeros((B, 1), jnp.float32)))

    out_ref[:, 0:128] = ob0
    out_ref[:, 128:256] = ob1
    out_ref[:, 256:384] = ob2
    out_ref[:, 384:512] = ob3
    out_ref[:, 512:640] = osc
    out_ref[:, 640:768] = ocl
    out_ref[:, 768:896] = jnp.broadcast_to(nv, (B, 128))
    out_ref[:, 896:1024] = zo


@jax.jit
def _run(predictions, anchors_t):
    B, N, C = _BATCH, _N_ANCH, _NUM_CLASSES
    padn = _N_PAD - N
    box_pred = jnp.transpose(predictions[:, :, :4], (0, 2, 1))
    box_pred = jnp.pad(box_pred, ((0, 0), (0, 0), (0, padn)))
    scores = jnp.transpose(predictions[:, :, 4:], (0, 2, 1))
    scores = jnp.pad(scores, ((0, 0), (0, 0), (0, padn)))
    scores = scores.reshape(_NPROB, _N_PAD)

    table = pl.pallas_call(
        _decode_kernel,
        grid=(B,),
        in_specs=[
            pl.BlockSpec((1, 4, _N_PAD), lambda i: (i, 0, 0)),
            pl.BlockSpec((4, _N_PAD), lambda i: (0, 0)),
        ],
        out_specs=pl.BlockSpec((1, 5, _N_PAD), lambda i: (i, 0, 0)),
        out_shape=jax.ShapeDtypeStruct((B, 5, _N_PAD), jnp.float32),
    )(box_pred, anchors_t)
    tflat = jnp.transpose(table, (1, 0, 2)).reshape(5, B * _N_PAD)

    mesh = plsc.VectorSubcoreMesh(core_axis_name="c", subcore_axis_name="s",
                                  num_cores=2, num_subcores=16)
    picks = pl.kernel(
        _sc_nms,
        out_type=jax.ShapeDtypeStruct((_NPROB, 6, 128), jnp.float32),
        mesh=mesh,
        compiler_params=pltpu.CompilerParams(needs_layout_passes=False),
        scratch_types=[
            pltpu.VMEM((_N_PAD,), jnp.float32),
            pltpu.VMEM((_CBUF,), jnp.float32),
            pltpu.VMEM((_CBUF,), jnp.int32),
            pltpu.VMEM((_CBUF,), jnp.float32),
            pltpu.VMEM((_CBUF,), jnp.float32),
            pltpu.VMEM((_CBUF,), jnp.float32),
            pltpu.VMEM((_CBUF,), jnp.float32),
            pltpu.VMEM((_CBUF,), jnp.float32),
            pltpu.VMEM((6, 128), jnp.float32),
            pltpu.SemaphoreType.DMA,
        ],
    )(scores, tflat[0], tflat[1], tflat[2], tflat[3], tflat[4])

    pt = jnp.transpose(picks, (1, 0, 2)).reshape(6, B, C * 128)

    out = pl.pallas_call(
        _merge_kernel,
        grid=(1,),
        in_specs=[pl.BlockSpec((B, C * 128), lambda i: (0, 0))] * 6,
        out_specs=pl.BlockSpec((B, 1024), lambda i: (0, 0)),
        out_shape=jax.ShapeDtypeStruct((B, 1024), jnp.float32),
    )(pt[0], pt[1], pt[2], pt[3], pt[4], pt[5])

    M = _MAX_TOTAL
    out3 = out.reshape(B, 8, 128)
    boxes = jnp.transpose(out3[:, 0:4, :M], (0, 2, 1))
    nmsed_scores = out3[:, 4, :M]
    nmsed_classes = out3[:, 5, :M]
    n_valid = out3[:, 6, 0].astype(jnp.int32)
    return boxes, nmsed_scores, nmsed_classes, n_valid


_ANCHORS_T = _anchors_t()


def kernel(predictions):
    return _run(predictions, _ANCHORS_T)


# speedup vs baseline: 30.1838x; 1.4216x over previous
import functools

import jax
import jax.numpy as jnp
import numpy as np
from jax import lax
from jax.experimental import pallas as pl
from jax.experimental.pallas import tpu as pltpu
from jax.experimental.pallas import tpu_sc as plsc

_STEPS = [32, 16, 8, 4, 2, 1]
_NUM_CLASSES = 20
_BATCH = 8
_N_ANCH = 9 * sum(n * n for n in _STEPS)
_N_PAD = 12288
_NPROB = _BATCH * _NUM_CLASSES
_IOU_THRESH = 0.5
_SCORE_THRESH = 0.03
_MAX_PER_CLASS = 100
_MAX_TOTAL = 100
_NEG = -1e9
_BIGI = 2**30
_KCAP1 = 256
_NV1 = _KCAP1 // 16
_NV2 = _N_PAD // 16
_CBUF = _N_PAD + 16


def _anchors_t() -> np.ndarray:
    all_b = []
    scales = [2.0 ** x for x in [0.0, 1.0 / 3.0, 2.0 / 3.0]]
    ratios = [0.5, 1.0, 2.0]
    for n in _STEPS:
        fw = 1.0 / n
        rows, cols = np.meshgrid(np.arange(n), np.arange(n), indexing='ij')
        cx = (cols + 0.5) * fw
        cy = (rows + 0.5) * fw
        whs = np.array(
            [[s * np.sqrt(r) * fw, s / np.sqrt(r) * fw]
             for s in scales for r in ratios], dtype=np.float64)
        cxcy = np.stack([cx, cy], axis=-1).reshape(n * n, 1, 2)
        loc = np.broadcast_to(cxcy, (n * n, 9, 2))
        whb = np.broadcast_to(whs.reshape(1, 9, 2), (n * n, 9, 2))
        all_b.append(np.concatenate([loc, whb], axis=-1).reshape(-1, 4))
    anch = np.concatenate(all_b, 0).astype(np.float32)
    pad = np.zeros((_N_PAD - _N_ANCH, 4), dtype=np.float32)
    return np.concatenate([anch, pad], 0).T.copy()


def _decode_kernel(bp_ref, anc_ref, out_ref):
    bp = bp_ref[0]
    anc = anc_ref[...]
    acx, acy, aw, ah = anc[0:1], anc[1:2], anc[2:3], anc[3:4]
    cx = bp[0:1] * 0.1 * aw + acx
    cy = bp[1:2] * 0.1 * ah + acy
    w = jnp.exp(bp[2:3] * 0.2) * aw
    h = jnp.exp(bp[3:4] * 0.2) * ah
    x1 = cx - w * 0.5
    y1 = cy - h * 0.5
    x2 = cx + w * 0.5
    y2 = cy + h * 0.5
    ar = (x2 - x1) * (y2 - y1)
    out_ref[0, :, :] = jnp.concatenate([x1, y1, x2, y2, ar], axis=0)


def _sc_nms(sc_ref, t0, t1, t2, t3, t4, out_ref,
            srow, cs, cidx, cx1, cy1, cx2, cy2, car, picks, sem):
    NC = 2
    wid = lax.axis_index("s") * NC + lax.axis_index("c")
    iota16 = lax.iota(jnp.int32, 16)
    negv = jnp.full((16,), _NEG, jnp.float32)

    def count_above(tau):
        def body(j, acc):
            v = srow[pl.ds(j * 16, 16)]
            return acc + (v > tau).astype(jnp.int32)
        acc = lax.fori_loop(0, _NV2, body, jnp.zeros((16,), jnp.int32))
        return jnp.sum(acc)

    def row_max():
        def body(j, m):
            return jnp.maximum(m, srow[pl.ds(j * 16, 16)])
        return jnp.max(lax.fori_loop(0, _NV2, body, negv))

    def prefill(nv, safe_idx):
        pad_i = jnp.full((16,), safe_idx, jnp.int32)
        def body(j, _):
            cs[pl.ds(j * 16, 16)] = negv
            cidx[pl.ds(j * 16, 16)] = pad_i
            return 0
        lax.fori_loop(0, nv + 1, body, 0)

    def compact(tau, base_idx):
        def body(j, cnt):
            v = srow[pl.ds(j * 16, 16)]
            m = v > tau
            plsc.store_compressed(cs.at[pl.ds(cnt, 16)], v, mask=m)
            gi = base_idx + j * 16 + iota16
            plsc.store_compressed(cidx.at[pl.ds(cnt, 16)], gi, mask=m)
            return cnt + jnp.sum(m.astype(jnp.int32))
        return lax.fori_loop(0, _NV2, body, 0)

    def gather_boxes(nchunks):
        for tbl, dst in ((t0, cx1), (t1, cy1), (t2, cx2), (t3, cy2),
                         (t4, car)):
            def body(q, _, tbl=tbl, dst=dst):
                idx = cidx.at[pl.ds(q * 128, 128)]
                pltpu.async_copy(tbl.at[idx], dst.at[pl.ds(q * 128, 128)],
                                 sem).wait()
                return 0
            lax.fori_loop(0, nchunks, body, 0)

    def greedy(nv, unroll=False):
        zeros16i = jnp.zeros((16,), jnp.int32)

        def am(j, carry):
            bm, bj = carry
            v = cs[pl.ds(j * 16, 16)]
            better = v > bm
            return (jnp.where(better, v, bm),
                    jnp.where(better, j, bj))
        if unroll:
            bm0, bj0 = negv, zeros16i
            for j in range(nv):
                bm0, bj0 = am(j, (bm0, bj0))
        else:
            bm0, bj0 = lax.fori_loop(0, nv, am, (negv, zeros16i))

        def w_cond(c):
            t, alive, _, _ = c
            return (t < _MAX_PER_CLASS) & alive

        def w_body(c):
            t, _, bm, bj = c
            bs = jnp.max(bm)
            alive = bs > _NEG / 2.0
            gi = jnp.where(bm == bs, bj * 16 + iota16, _BIGI)
            pos = jnp.min(gi)

            def do_pick():
                b0 = cx1[pl.ds(pos, 16)][0]
                b1 = cy1[pl.ds(pos, 16)][0]
                b2 = cx2[pl.ds(pos, 16)][0]
                b3 = cy2[pl.ds(pos, 16)][0]
                ba = car[pl.ds(pos, 16)][0]

                def su(j, carry):
                    nbm, nbj = carry
                    v = cs[pl.ds(j * 16, 16)]
                    vx1 = cx1[pl.ds(j * 16, 16)]
                    vy1 = cy1[pl.ds(j * 16, 16)]
                    vx2 = cx2[pl.ds(j * 16, 16)]
                    vy2 = cy2[pl.ds(j * 16, 16)]
                    va = car[pl.ds(j * 16, 16)]
                    ix1 = jnp.maximum(b0, vx1)
                    iy1 = jnp.maximum(b1, vy1)
                    ix2 = jnp.minimum(b2, vx2)
                    iy2 = jnp.minimum(b3, vy2)
                    inter = (jnp.maximum(ix2 - ix1, 0.0)
                             * jnp.maximum(iy2 - iy1, 0.0))
                    iou = inter / (va + ba - inter + 1e-8)
                    supp = (iou >= _IOU_THRESH) | (j * 16 + iota16 == pos)
                    newv = jnp.where(supp, _NEG, v)
                    cs[pl.ds(j * 16, 16)] = newv
                    better = newv > nbm
                    return (jnp.where(better, newv, nbm),
                            jnp.where(better, j, nbj))
                if unroll:
                    nbm, nbj = negv, zeros16i
                    for j in range(nv):
                        nbm, nbj = su(j, (nbm, nbj))
                else:
                    nbm, nbj = lax.fori_loop(0, nv, su, (negv, zeros16i))
                colv = jnp.full((16,), t, jnp.int32)
                lane0 = iota16 == 0
                for r, val in enumerate((b0, b1, b2, b3, bs,
                                         jnp.float32(1.0))):
                    plsc.store_scatter(
                        picks, [jnp.full((16,), r, jnp.int32), colv],
                        jnp.full((16,), val, jnp.float32), mask=lane0)
                return nbm, nbj

            def no_pick():
                return bm, bj

            nbm, nbj = lax.cond(alive, do_pick, no_pick)
            return t + alive.astype(jnp.int32), alive, nbm, nbj

        t, _, _, _ = lax.while_loop(w_cond, w_body, (0, True, bm0, bj0))
        return t

    def problem(k, _):
        p = wid * 5 + k
        img = p // _NUM_CLASSES
        base_idx = img * _N_PAD
        pltpu.sync_copy(sc_ref.at[p], srow)
        for r in range(6):
            for q in range(8):
                picks[r, pl.ds(q * 16, 16)] = jnp.zeros((16,), jnp.float32)

        c03 = count_above(jnp.float32(_SCORE_THRESH))

        def solve_dense():
            prefill(_NV2, base_idx)
            compact(jnp.float32(_SCORE_THRESH), base_idx)
            gather_boxes(_N_PAD // 128)
            greedy(_NV2)

        def solve_small():
            prefill(_NV1, base_idx)
            compact(jnp.float32(_SCORE_THRESH), base_idx)
            gather_boxes(_KCAP1 // 128)
            greedy(_NV1, unroll=True)

        def solve_truncated():
            smax = row_max()

            def b_cond(c):
                lo, hi, tau, cnt, it = c
                return ((cnt < _MAX_PER_CLASS) | (cnt > _KCAP1)) & (it < 24)

            def b_body(c):
                lo, hi, tau, cnt, it = c
                mid = 0.5 * (lo + hi)
                first = jnp.float32(2.20)
                mid = jnp.where((it == 0) & (first > lo) & (first < hi),
                                first, mid)
                cc = count_above(mid)
                lo = jnp.where(cc > _KCAP1, mid, lo)
                hi = jnp.where(cc > _KCAP1, hi, mid)
                return lo, hi, mid, cc, it + 1

            lo0 = jnp.float32(_SCORE_THRESH)
            _, _, tau, cnt, _ = lax.while_loop(
                b_cond, b_body, (lo0, smax, lo0, c03, 0))

            def trunc_path():
                prefill(_NV1, base_idx)
                compact(tau, base_idx)
                gather_boxes(_KCAP1 // 128)
                npicks = greedy(_NV1, unroll=True)
                pl.when(npicks < _MAX_PER_CLASS)(solve_dense)

            lax.cond(cnt > _KCAP1, solve_dense, trunc_path)

        lax.cond(c03 <= _KCAP1, solve_small, solve_truncated)
        pltpu.sync_copy(picks, out_ref.at[p])
        return 0

    lax.fori_loop(0, _NPROB // 32, problem, 0)


def _merge_kernel(x1_ref, y1_ref, x2_ref, y2_ref, sc_ref, vd_ref, out_ref):
    B, W = _BATCH, _NUM_CLASSES * 128
    sx1, sy1 = x1_ref[...], y1_ref[...]
    sx2, sy2 = x2_ref[...], y2_ref[...]
    ssc, svd = sc_ref[...], vd_ref[...]

    key = jnp.where(svd > 0.5, ssc, _NEG)
    fmat = lax.broadcasted_iota(jnp.int32, (B, W), 1)
    out_iota = lax.broadcasted_iota(jnp.int32, (B, 128), 1)
    zo = jnp.zeros((B, 128), jnp.float32)

    def merge_step(t2, carry):
        key, ob0, ob1, ob2, ob3, osc, ocl, nv = carry
        mv = jnp.max(key, axis=1, keepdims=True)
        eq2 = key == mv
        bf = jnp.min(jnp.where(eq2, fmat, _BIGI), axis=1,
                     keepdims=True)
        oh2 = fmat == bf
        ninf = jnp.float32(-3e38)
        g0 = jnp.max(jnp.where(oh2, sx1, ninf), axis=1, keepdims=True)
        g1 = jnp.max(jnp.where(oh2, sy1, ninf), axis=1, keepdims=True)
        g2 = jnp.max(jnp.where(oh2, sx2, ninf), axis=1, keepdims=True)
        g3 = jnp.max(jnp.where(oh2, sy2, ninf), axis=1, keepdims=True)
        vflag = mv > _NEG / 2.0
        clsf = (bf // 128).astype(jnp.float32)
        vz = jnp.float32(0.0)
        c0 = jnp.where(vflag, jnp.clip(g0, 0.0, 1.0), vz)
        c1 = jnp.where(vflag, jnp.clip(g1, 0.0, 1.0), vz)
        c2 = jnp.where(vflag, jnp.clip(g2, 0.0, 1.0), vz)
        c3 = jnp.where(vflag, jnp.clip(g3, 0.0, 1.0), vz)
        cs = jnp.where(vflag, mv, vz)
        cc = jnp.where(vflag, clsf, vz)
        sel2 = out_iota == t2
        ob0 = jnp.where(sel2, c0, ob0)
        ob1 = jnp.where(sel2, c1, ob1)
        ob2 = jnp.where(sel2, c2, ob2)
        ob3 = jnp.where(sel2, c3, ob3)
        osc = jnp.where(sel2, cs, osc)
        ocl = jnp.where(sel2, cc, ocl)
        nv = nv + jnp.where(vflag, 1.0, 0.0)
        key = jnp.where(oh2, jnp.float32(-3e38), key)
        return key, ob0, ob1, ob2, ob3, osc, ocl, nv

    _, ob0, ob1, ob2, ob3, osc, ocl, nv = lax.fori_loop(
        0, _MAX_TOTAL, merge_step,
        (key, zo, zo, zo, zo, zo, zo, jnp.zeros((B, 1), jnp.float32)))

    out_ref[:, 0:128] = ob0
    out_ref[:, 128:256] = ob1
    out_ref[:, 256:384] = ob2
    out_ref[:, 384:512] = ob3
    out_ref[:, 512:640] = osc
    out_ref[:, 640:768] = ocl
    out_ref[:, 768:896] = jnp.broadcast_to(nv, (B, 128))
    out_ref[:, 896:1024] = zo


@jax.jit
def _run(predictions, anchors_t):
    B, N, C = _BATCH, _N_ANCH, _NUM_CLASSES
    padn = _N_PAD - N
    box_pred = jnp.transpose(predictions[:, :, :4], (0, 2, 1))
    box_pred = jnp.pad(box_pred, ((0, 0), (0, 0), (0, padn)))
    scores = jnp.transpose(predictions[:, :, 4:], (0, 2, 1))
    scores = jnp.pad(scores, ((0, 0), (0, 0), (0, padn)))
    scores = scores.reshape(_NPROB, _N_PAD)

    table = pl.pallas_call(
        _decode_kernel,
        grid=(B,),
        in_specs=[
            pl.BlockSpec((1, 4, _N_PAD), lambda i: (i, 0, 0)),
            pl.BlockSpec((4, _N_PAD), lambda i: (0, 0)),
        ],
        out_specs=pl.BlockSpec((1, 5, _N_PAD), lambda i: (i, 0, 0)),
        out_shape=jax.ShapeDtypeStruct((B, 5, _N_PAD), jnp.float32),
    )(box_pred, anchors_t)
    tflat = jnp.transpose(table, (1, 0, 2)).reshape(5, B * _N_PAD)

    mesh = plsc.VectorSubcoreMesh(core_axis_name="c", subcore_axis_name="s",
                                  num_cores=2, num_subcores=16)
    picks = pl.kernel(
        _sc_nms,
        out_type=jax.ShapeDtypeStruct((_NPROB, 6, 128), jnp.float32),
        mesh=mesh,
        compiler_params=pltpu.CompilerParams(needs_layout_passes=False),
        scratch_types=[
            pltpu.VMEM((_N_PAD,), jnp.float32),
            pltpu.VMEM((_CBUF,), jnp.float32),
            pltpu.VMEM((_CBUF,), jnp.int32),
            pltpu.VMEM((_CBUF,), jnp.float32),
            pltpu.VMEM((_CBUF,), jnp.float32),
            pltpu.VMEM((_CBUF,), jnp.float32),
            pltpu.VMEM((_CBUF,), jnp.float32),
            pltpu.VMEM((_CBUF,), jnp.float32),
            pltpu.VMEM((6, 128), jnp.float32),
            pltpu.SemaphoreType.DMA,
        ],
    )(scores, tflat[0], tflat[1], tflat[2], tflat[3], tflat[4])

    pt = jnp.transpose(picks, (1, 0, 2)).reshape(6, B, C * 128)

    out = pl.pallas_call(
        _merge_kernel,
        grid=(1,),
        in_specs=[pl.BlockSpec((B, C * 128), lambda i: (0, 0))] * 6,
        out_specs=pl.BlockSpec((B, 1024), lambda i: (0, 0)),
        out_shape=jax.ShapeDtypeStruct((B, 1024), jnp.float32),
    )(pt[0], pt[1], pt[2], pt[3], pt[4], pt[5])

    M = _MAX_TOTAL
    out3 = out.reshape(B, 8, 128)
    boxes = jnp.transpose(out3[:, 0:4, :M], (0, 2, 1))
    nmsed_scores = out3[:, 4, :M]
    nmsed_classes = out3[:, 5, :M]
    n_valid = out3[:, 6, 0].astype(jnp.int32)
    return boxes, nmsed_scores, nmsed_classes, n_valid


_ANCHORS_T = _anchors_t()


def kernel(predictions):
    return _run(predictions, _ANCHORS_T)


# speedup vs baseline: 33.8287x; 1.1208x over previous
import functools

import jax
import jax.numpy as jnp
import numpy as np
from jax import lax
from jax.experimental import pallas as pl
from jax.experimental.pallas import tpu as pltpu
from jax.experimental.pallas import tpu_sc as plsc

_STEPS = [32, 16, 8, 4, 2, 1]
_NUM_CLASSES = 20
_BATCH = 8
_N_ANCH = 9 * sum(n * n for n in _STEPS)
_N_PAD = 12288
_NPROB = _BATCH * _NUM_CLASSES
_IOU_THRESH = 0.5
_SCORE_THRESH = 0.03
_MAX_PER_CLASS = 100
_MAX_TOTAL = 100
_NEG = -1e9
_BIGI = 2**30
_KCAP1 = 256
_NV1 = _KCAP1 // 16
_NV2 = _N_PAD // 16
_CBUF = _N_PAD + 16


def _anchors_t() -> np.ndarray:
    all_b = []
    scales = [2.0 ** x for x in [0.0, 1.0 / 3.0, 2.0 / 3.0]]
    ratios = [0.5, 1.0, 2.0]
    for n in _STEPS:
        fw = 1.0 / n
        rows, cols = np.meshgrid(np.arange(n), np.arange(n), indexing='ij')
        cx = (cols + 0.5) * fw
        cy = (rows + 0.5) * fw
        whs = np.array(
            [[s * np.sqrt(r) * fw, s / np.sqrt(r) * fw]
             for s in scales for r in ratios], dtype=np.float64)
        cxcy = np.stack([cx, cy], axis=-1).reshape(n * n, 1, 2)
        loc = np.broadcast_to(cxcy, (n * n, 9, 2))
        whb = np.broadcast_to(whs.reshape(1, 9, 2), (n * n, 9, 2))
        all_b.append(np.concatenate([loc, whb], axis=-1).reshape(-1, 4))
    anch = np.concatenate(all_b, 0).astype(np.float32)
    pad = np.zeros((_N_PAD - _N_ANCH, 4), dtype=np.float32)
    return np.concatenate([anch, pad], 0).T.copy()


def _decode_kernel(bp_ref, anc_ref, out_ref):
    bp = bp_ref[0]
    anc = anc_ref[...]
    acx, acy, aw, ah = anc[0:1], anc[1:2], anc[2:3], anc[3:4]
    cx = bp[0:1] * 0.1 * aw + acx
    cy = bp[1:2] * 0.1 * ah + acy
    w = jnp.exp(bp[2:3] * 0.2) * aw
    h = jnp.exp(bp[3:4] * 0.2) * ah
    x1 = cx - w * 0.5
    y1 = cy - h * 0.5
    x2 = cx + w * 0.5
    y2 = cy + h * 0.5
    ar = (x2 - x1) * (y2 - y1)
    out_ref[0, :, :] = jnp.concatenate([x1, y1, x2, y2, ar], axis=0)


def _sc_nms(sc_ref, t0, t1, t2, t3, t4, shared, out_ref,
            srow, cs, cidx, cx1, cy1, cx2, cy2, car, picks, obuf, sem):
    cid = lax.axis_index("c")
    sid = lax.axis_index("s")
    iota16 = lax.iota(jnp.int32, 16)
    negv = jnp.full((16,), _NEG, jnp.float32)

    def count_above(tau):
        def body(j, acc):
            v = srow[pl.ds(j * 16, 16)]
            return acc + (v > tau).astype(jnp.int32)
        acc = lax.fori_loop(0, _NV2, body, jnp.zeros((16,), jnp.int32))
        return jnp.sum(acc)

    def row_max():
        def body(j, m):
            return jnp.maximum(m, srow[pl.ds(j * 16, 16)])
        return jnp.max(lax.fori_loop(0, _NV2, body, negv))

    def prefill(nv, safe_idx):
        pad_i = jnp.full((16,), safe_idx, jnp.int32)
        def body(j, _):
            cs[pl.ds(j * 16, 16)] = negv
            cidx[pl.ds(j * 16, 16)] = pad_i
            return 0
        lax.fori_loop(0, nv + 1, body, 0)

    def compact(tau, base_idx):
        def body(j, cnt):
            v = srow[pl.ds(j * 16, 16)]
            m = v > tau
            plsc.store_compressed(cs.at[pl.ds(cnt, 16)], v, mask=m)
            gi = base_idx + j * 16 + iota16
            plsc.store_compressed(cidx.at[pl.ds(cnt, 16)], gi, mask=m)
            return cnt + jnp.sum(m.astype(jnp.int32))
        return lax.fori_loop(0, _NV2, body, 0)

    def gather_boxes(nchunks):
        for tbl, dst in ((t0, cx1), (t1, cy1), (t2, cx2), (t3, cy2),
                         (t4, car)):
            def body(q, _, tbl=tbl, dst=dst):
                idx = cidx.at[pl.ds(q * 128, 128)]
                pltpu.async_copy(tbl.at[idx], dst.at[pl.ds(q * 128, 128)],
                                 sem).wait()
                return 0
            lax.fori_loop(0, nchunks, body, 0)

    def greedy(nv, unroll=False):
        zeros16i = jnp.zeros((16,), jnp.int32)

        def am(j, carry):
            bm, bj = carry
            v = cs[pl.ds(j * 16, 16)]
            better = v > bm
            return (jnp.where(better, v, bm),
                    jnp.where(better, j, bj))
        if unroll:
            bm0, bj0 = negv, zeros16i
            for j in range(nv):
                bm0, bj0 = am(j, (bm0, bj0))
        else:
            bm0, bj0 = lax.fori_loop(0, nv, am, (negv, zeros16i))

        def w_cond(c):
            t, alive, _, _ = c
            return (t < _MAX_PER_CLASS) & alive

        def w_body(c):
            t, _, bm, bj = c
            bs = jnp.max(bm)
            alive = bs > _NEG / 2.0
            gi = jnp.where(bm == bs, bj * 16 + iota16, _BIGI)
            pos = jnp.min(gi)

            def do_pick():
                b0 = cx1[pl.ds(pos, 16)][0]
                b1 = cy1[pl.ds(pos, 16)][0]
                b2 = cx2[pl.ds(pos, 16)][0]
                b3 = cy2[pl.ds(pos, 16)][0]
                ba = car[pl.ds(pos, 16)][0]

                def su(j, carry):
                    nbm, nbj = carry
                    v = cs[pl.ds(j * 16, 16)]
                    vx1 = cx1[pl.ds(j * 16, 16)]
                    vy1 = cy1[pl.ds(j * 16, 16)]
                    vx2 = cx2[pl.ds(j * 16, 16)]
                    vy2 = cy2[pl.ds(j * 16, 16)]
                    va = car[pl.ds(j * 16, 16)]
                    ix1 = jnp.maximum(b0, vx1)
                    iy1 = jnp.maximum(b1, vy1)
                    ix2 = jnp.minimum(b2, vx2)
                    iy2 = jnp.minimum(b3, vy2)
                    inter = (jnp.maximum(ix2 - ix1, 0.0)
                             * jnp.maximum(iy2 - iy1, 0.0))
                    iou = inter / (va + ba - inter + 1e-8)
                    supp = (iou >= _IOU_THRESH) | (j * 16 + iota16 == pos)
                    newv = jnp.where(supp, _NEG, v)
                    cs[pl.ds(j * 16, 16)] = newv
                    better = newv > nbm
                    return (jnp.where(better, newv, nbm),
                            jnp.where(better, j, nbj))
                if unroll:
                    nbm, nbj = negv, zeros16i
                    for j in range(nv):
                        nbm, nbj = su(j, (nbm, nbj))
                else:
                    nbm, nbj = lax.fori_loop(0, nv, su, (negv, zeros16i))
                colv = jnp.full((16,), t, jnp.int32)
                lane0 = iota16 == 0
                for r, val in enumerate((b0, b1, b2, b3, bs,
                                         jnp.float32(1.0))):
                    plsc.store_scatter(
                        picks, [jnp.full((16,), r, jnp.int32), colv],
                        jnp.full((16,), val, jnp.float32), mask=lane0)
                return nbm, nbj

            def no_pick():
                return bm, bj

            nbm, nbj = lax.cond(alive, do_pick, no_pick)
            return t + alive.astype(jnp.int32), alive, nbm, nbj

        t, _, _, _ = lax.while_loop(w_cond, w_body, (0, True, bm0, bj0))
        return t

    def problem(k, _):
        q = sid * 5 + k
        p = cid * 80 + q
        img = p // _NUM_CLASSES
        base_idx = img * _N_PAD
        pltpu.sync_copy(sc_ref.at[p], srow)
        for r in range(6):
            for q in range(8):
                picks[r, pl.ds(q * 16, 16)] = jnp.zeros((16,), jnp.float32)

        c03 = count_above(jnp.float32(_SCORE_THRESH))

        def solve_dense():
            prefill(_NV2, base_idx)
            compact(jnp.float32(_SCORE_THRESH), base_idx)
            gather_boxes(_N_PAD // 128)
            greedy(_NV2)

        def solve_small():
            prefill(_NV1, base_idx)
            compact(jnp.float32(_SCORE_THRESH), base_idx)
            gather_boxes(_KCAP1 // 128)
            greedy(_NV1, unroll=True)

        def solve_truncated():
            smax = row_max()

            def b_cond(c):
                lo, hi, tau, cnt, it = c
                return ((cnt < _MAX_PER_CLASS) | (cnt > _KCAP1)) & (it < 24)

            def b_body(c):
                lo, hi, tau, cnt, it = c
                mid = 0.5 * (lo + hi)
                first = jnp.float32(2.20)
                mid = jnp.where((it == 0) & (first > lo) & (first < hi),
                                first, mid)
                cc = count_above(mid)
                lo = jnp.where(cc > _KCAP1, mid, lo)
                hi = jnp.where(cc > _KCAP1, hi, mid)
                return lo, hi, mid, cc, it + 1

            lo0 = jnp.float32(_SCORE_THRESH)
            _, _, tau, cnt, _ = lax.while_loop(
                b_cond, b_body, (lo0, smax, lo0, c03, 0))

            def trunc_path():
                prefill(_NV1, base_idx)
                compact(tau, base_idx)
                gather_boxes(_KCAP1 // 128)
                npicks = greedy(_NV1, unroll=True)
                pl.when(npicks < _MAX_PER_CLASS)(solve_dense)

            lax.cond(cnt > _KCAP1, solve_dense, trunc_path)

        lax.cond(c03 <= _KCAP1, solve_small, solve_truncated)
        for r in range(6):
            pltpu.sync_copy(picks.at[r], shared.at[r, p])
        return 0

    lax.fori_loop(0, _NPROB // 32, problem, 0)
    plsc.subcore_barrier()

    @pl.when(sid < 4)
    def _merge():
        img = cid * 4 + sid
        copies = []
        for src_r, dst in ((0, cx1), (1, cy1), (2, cx2), (3, cy2),
                           (4, cs), (5, srow)):
            for j in range(_NUM_CLASSES):
                copies.append(pltpu.async_copy(
                    shared.at[src_r, img * _NUM_CLASSES + j],
                    dst.at[pl.ds(j * 128, 16 * 8)], sem))
        for h in copies:
            h.wait()
        def mk(j, _):
            sv = srow[pl.ds(j * 16, 16)]
            scv = cs[pl.ds(j * 16, 16)]
            cs[pl.ds(j * 16, 16)] = jnp.where(sv > 0.5, scv, _NEG)
            return 0
        lax.fori_loop(0, _NUM_CLASSES * 8, mk, 0)
        for j2 in range(64):
            obuf[pl.ds(j2 * 16, 16)] = jnp.zeros((16,), jnp.float32)

        h0 = plsc.load_gather(cs, [iota16 * 128])
        h1r = plsc.load_gather(cs, [(iota16 + 16) * 128])
        h1 = jnp.where(iota16 < _NUM_CLASSES - 16, h1r, _NEG)
        zi = jnp.zeros((16,), jnp.int32)

        def mstep(t2, carry):
            h0, h1, cu0, cu1, nv = carry
            bs = jnp.max(jnp.maximum(h0, h1))
            valid = bs > _NEG / 2.0
            cand = jnp.minimum(jnp.where(h0 == bs, iota16, _BIGI),
                               jnp.where(h1 == bs, iota16 + 16, _BIGI))
            cstar = jnp.min(cand)
            tcv = jnp.minimum(jnp.where(iota16 == cstar, cu0, _BIGI),
                              jnp.where(iota16 + 16 == cstar, cu1, _BIGI))
            tc = jnp.min(tcv)
            pos = cstar * 128 + tc

            @pl.when(valid)
            def _():
                b0 = jnp.clip(cx1[pl.ds(pos, 16)][0], 0.0, 1.0)
                b1 = jnp.clip(cy1[pl.ds(pos, 16)][0], 0.0, 1.0)
                b2 = jnp.clip(cx2[pl.ds(pos, 16)][0], 0.0, 1.0)
                b3 = jnp.clip(cy2[pl.ds(pos, 16)][0], 0.0, 1.0)
                clsf = cstar.astype(jnp.float32)
                x = jnp.where(iota16 == 0, b0,
                    jnp.where(iota16 == 1, b1,
                    jnp.where(iota16 == 2, b2,
                    jnp.where(iota16 == 3, b3,
                    jnp.where(iota16 == 4, bs, clsf)))))
                plsc.store_scatter(obuf, [iota16 * 128 + t2], x,
                                   mask=iota16 < 6)

            hv = cs[pl.ds(pos + 1, 16)][0]
            adv0 = valid & (iota16 == cstar)
            adv1 = valid & (iota16 + 16 == cstar)
            h0n = jnp.where(adv0, hv, h0)
            h1n = jnp.where(adv1, hv, h1)
            cu0n = jnp.where(adv0, tc + 1, cu0)
            cu1n = jnp.where(adv1, tc + 1, cu1)
            return h0n, h1n, cu0n, cu1n, nv + valid.astype(jnp.int32)

        _, _, _, _, nv = lax.fori_loop(
            0, _MAX_TOTAL, mstep, (h0, h1, zi, zi, 0))
        nvf = jnp.full((16,), nv.astype(jnp.float32))
        for j3 in range(8):
            obuf[pl.ds(6 * 128 + j3 * 16, 16)] = nvf
        pltpu.sync_copy(obuf, out_ref.at[img])


def _merge_kernel(x1_ref, y1_ref, x2_ref, y2_ref, sc_ref, vd_ref, out_ref):
    B, W = _BATCH, _NUM_CLASSES * 128
    sx1, sy1 = x1_ref[...], y1_ref[...]
    sx2, sy2 = x2_ref[...], y2_ref[...]
    ssc, svd = sc_ref[...], vd_ref[...]

    key = jnp.where(svd > 0.5, ssc, _NEG)
    fmat = lax.broadcasted_iota(jnp.int32, (B, W), 1)
    out_iota = lax.broadcasted_iota(jnp.int32, (B, 128), 1)
    zo = jnp.zeros((B, 128), jnp.float32)

    def merge_step(t2, carry):
        key, ob0, ob1, ob2, ob3, osc, ocl, nv = carry
        mv = jnp.max(key, axis=1, keepdims=True)
        eq2 = key == mv
        bf = jnp.min(jnp.where(eq2, fmat, _BIGI), axis=1,
                     keepdims=True)
        oh2 = fmat == bf
        ninf = jnp.float32(-3e38)
        g0 = jnp.max(jnp.where(oh2, sx1, ninf), axis=1, keepdims=True)
        g1 = jnp.max(jnp.where(oh2, sy1, ninf), axis=1, keepdims=True)
        g2 = jnp.max(jnp.where(oh2, sx2, ninf), axis=1, keepdims=True)
        g3 = jnp.max(jnp.where(oh2, sy2, ninf), axis=1, keepdims=True)
        vflag = mv > _NEG / 2.0
        clsf = (bf // 128).astype(jnp.float32)
        vz = jnp.float32(0.0)
        c0 = jnp.where(vflag, jnp.clip(g0, 0.0, 1.0), vz)
        c1 = jnp.where(vflag, jnp.clip(g1, 0.0, 1.0), vz)
        c2 = jnp.where(vflag, jnp.clip(g2, 0.0, 1.0), vz)
        c3 = jnp.where(vflag, jnp.clip(g3, 0.0, 1.0), vz)
        cs = jnp.where(vflag, mv, vz)
        cc = jnp.where(vflag, clsf, vz)
        sel2 = out_iota == t2
        ob0 = jnp.where(sel2, c0, ob0)
        ob1 = jnp.where(sel2, c1, ob1)
        ob2 = jnp.where(sel2, c2, ob2)
        ob3 = jnp.where(sel2, c3, ob3)
        osc = jnp.where(sel2, cs, osc)
        ocl = jnp.where(sel2, cc, ocl)
        nv = nv + jnp.where(vflag, 1.0, 0.0)
        key = jnp.where(oh2, jnp.float32(-3e38), key)
        return key, ob0, ob1, ob2, ob3, osc, ocl, nv

    _, ob0, ob1, ob2, ob3, osc, ocl, nv = lax.fori_loop(
        0, _MAX_TOTAL, merge_step,
        (key, zo, zo, zo, zo, zo, zo, jnp.zeros((B, 1), jnp.float32)))

    out_ref[:, 0:128] = ob0
    out_ref[:, 128:256] = ob1
    out_ref[:, 256:384] = ob2
    out_ref[:, 384:512] = ob3
    out_ref[:, 512:640] = osc
    out_ref[:, 640:768] = ocl
    out_ref[:, 768:896] = jnp.broadcast_to(nv, (B, 128))
    out_ref[:, 896:1024] = zo


@jax.jit
def _run(predictions, anchors_t):
    B, N, C = _BATCH, _N_ANCH, _NUM_CLASSES
    padn = _N_PAD - N
    box_pred = jnp.transpose(predictions[:, :, :4], (0, 2, 1))
    box_pred = jnp.pad(box_pred, ((0, 0), (0, 0), (0, padn)))
    scores = jnp.transpose(predictions[:, :, 4:], (0, 2, 1))
    scores = jnp.pad(scores, ((0, 0), (0, 0), (0, padn)))
    scores = scores.reshape(_NPROB, _N_PAD)

    table = pl.pallas_call(
        _decode_kernel,
        grid=(B,),
        in_specs=[
            pl.BlockSpec((1, 4, _N_PAD), lambda i: (i, 0, 0)),
            pl.BlockSpec((4, _N_PAD), lambda i: (0, 0)),
        ],
        out_specs=pl.BlockSpec((1, 5, _N_PAD), lambda i: (i, 0, 0)),
        out_shape=jax.ShapeDtypeStruct((B, 5, _N_PAD), jnp.float32),
    )(box_pred, anchors_t)
    tflat = jnp.transpose(table, (1, 0, 2)).reshape(5, B * _N_PAD)

    mesh = plsc.VectorSubcoreMesh(core_axis_name="c", subcore_axis_name="s",
                                  num_cores=2, num_subcores=16)
    _, out = pl.kernel(
        _sc_nms,
        out_type=(jax.ShapeDtypeStruct((6, _NPROB, 128), jnp.float32),
                  jax.ShapeDtypeStruct((B, 1024), jnp.float32)),
        mesh=mesh,
        compiler_params=pltpu.CompilerParams(needs_layout_passes=False),
        scratch_types=[
            pltpu.VMEM((_N_PAD,), jnp.float32),
            pltpu.VMEM((_CBUF,), jnp.float32),
            pltpu.VMEM((_CBUF,), jnp.int32),
            pltpu.VMEM((_CBUF,), jnp.float32),
            pltpu.VMEM((_CBUF,), jnp.float32),
            pltpu.VMEM((_CBUF,), jnp.float32),
            pltpu.VMEM((_CBUF,), jnp.float32),
            pltpu.VMEM((_CBUF,), jnp.float32),
            pltpu.VMEM((6, 128), jnp.float32),
            pltpu.VMEM((1024,), jnp.float32),
            pltpu.SemaphoreType.DMA,
        ],
    )(scores, tflat[0], tflat[1], tflat[2], tflat[3], tflat[4])

    M = _MAX_TOTAL
    out3 = out.reshape(B, 8, 128)
    boxes = jnp.transpose(out3[:, 0:4, :M], (0, 2, 1))
    nmsed_scores = out3[:, 4, :M]
    nmsed_classes = out3[:, 5, :M]
    n_valid = out3[:, 6, 0].astype(jnp.int32)
    return boxes, nmsed_scores, nmsed_classes, n_valid


_ANCHORS_T = _anchors_t()


def kernel(predictions):
    return _run(predictions, _ANCHORS_T)


# speedup vs baseline: 42.3323x; 1.2514x over previous
import functools

import jax
import jax.numpy as jnp
import numpy as np
from jax import lax
from jax.experimental import pallas as pl
from jax.experimental.pallas import tpu as pltpu
from jax.experimental.pallas import tpu_sc as plsc

_STEPS = [32, 16, 8, 4, 2, 1]
_NUM_CLASSES = 20
_BATCH = 8
_N_ANCH = 9 * sum(n * n for n in _STEPS)
_N_PAD = 12288
_NPROB = _BATCH * _NUM_CLASSES
_IOU_THRESH = 0.5
_SCORE_THRESH = 0.03
_MAX_PER_CLASS = 100
_MAX_TOTAL = 100
_NEG = -1e9
_BIGI = 2**30
_KCAP1 = 192
_NV1 = _KCAP1 // 16
_NV2 = _N_PAD // 16
_CBUF = _N_PAD + 16


def _anchors_t() -> np.ndarray:
    all_b = []
    scales = [2.0 ** x for x in [0.0, 1.0 / 3.0, 2.0 / 3.0]]
    ratios = [0.5, 1.0, 2.0]
    for n in _STEPS:
        fw = 1.0 / n
        rows, cols = np.meshgrid(np.arange(n), np.arange(n), indexing='ij')
        cx = (cols + 0.5) * fw
        cy = (rows + 0.5) * fw
        whs = np.array(
            [[s * np.sqrt(r) * fw, s / np.sqrt(r) * fw]
             for s in scales for r in ratios], dtype=np.float64)
        cxcy = np.stack([cx, cy], axis=-1).reshape(n * n, 1, 2)
        loc = np.broadcast_to(cxcy, (n * n, 9, 2))
        whb = np.broadcast_to(whs.reshape(1, 9, 2), (n * n, 9, 2))
        all_b.append(np.concatenate([loc, whb], axis=-1).reshape(-1, 4))
    anch = np.concatenate(all_b, 0).astype(np.float32)
    pad = np.zeros((_N_PAD - _N_ANCH, 4), dtype=np.float32)
    return np.concatenate([anch, pad], 0).T.copy()


def _decode_kernel(bp_ref, anc_ref, out_ref):
    bp = bp_ref[0]
    anc = anc_ref[...]
    acx, acy, aw, ah = anc[0:1], anc[1:2], anc[2:3], anc[3:4]
    cx = bp[0:1] * 0.1 * aw + acx
    cy = bp[1:2] * 0.1 * ah + acy
    w = jnp.exp(bp[2:3] * 0.2) * aw
    h = jnp.exp(bp[3:4] * 0.2) * ah
    x1 = cx - w * 0.5
    y1 = cy - h * 0.5
    x2 = cx + w * 0.5
    y2 = cy + h * 0.5
    ar = (x2 - x1) * (y2 - y1)
    out_ref[0, :, :] = jnp.concatenate([x1, y1, x2, y2, ar], axis=0)


def _sc_nms(sc_ref, t0, t1, t2, t3, t4, shared, out_ref,
            srow, cs, cidx, cx1, cy1, cx2, cy2, car, picks, obuf, sem):
    cid = lax.axis_index("c")
    sid = lax.axis_index("s")
    iota16 = lax.iota(jnp.int32, 16)
    negv = jnp.full((16,), _NEG, jnp.float32)

    def count_above(tau):
        def body(j, acc):
            v = srow[pl.ds(j * 16, 16)]
            return acc + (v > tau).astype(jnp.int32)
        acc = lax.fori_loop(0, _NV2, body, jnp.zeros((16,), jnp.int32))
        return jnp.sum(acc)

    def prefill(nv, safe_idx):
        pad_i = jnp.full((16,), safe_idx, jnp.int32)
        def body(j, _):
            cs[pl.ds(j * 16, 16)] = negv
            cidx[pl.ds(j * 16, 16)] = pad_i
            return 0
        lax.fori_loop(0, nv + 1, body, 0)

    def compact(tau, base_idx):
        def body(j, cnt):
            v = srow[pl.ds(j * 16, 16)]
            m = v > tau
            plsc.store_compressed(cs.at[pl.ds(cnt, 16)], v, mask=m)
            gi = base_idx + j * 16 + iota16
            plsc.store_compressed(cidx.at[pl.ds(cnt, 16)], gi, mask=m)
            return cnt + jnp.sum(m.astype(jnp.int32))
        return lax.fori_loop(0, _NV2, body, 0)

    def gather_boxes(nchunks):
        for tbl, dst in ((t0, cx1), (t1, cy1), (t2, cx2), (t3, cy2),
                         (t4, car)):
            def body(q, _, tbl=tbl, dst=dst):
                idx = cidx.at[pl.ds(q * 128, 128)]
                pltpu.async_copy(tbl.at[idx], dst.at[pl.ds(q * 128, 128)],
                                 sem).wait()
                return 0
            lax.fori_loop(0, nchunks, body, 0)

    def greedy(nv, unroll=False):
        zeros16i = jnp.zeros((16,), jnp.int32)

        def am(j, carry):
            bm, bj = carry
            v = cs[pl.ds(j * 16, 16)]
            better = v > bm
            return (jnp.where(better, v, bm),
                    jnp.where(better, j, bj))
        if unroll:
            bm0, bj0 = negv, zeros16i
            for j in range(nv):
                bm0, bj0 = am(j, (bm0, bj0))
        else:
            bm0, bj0 = lax.fori_loop(0, nv, am, (negv, zeros16i))

        def w_cond(c):
            t, alive, _, _ = c
            return (t < _MAX_PER_CLASS) & alive

        def w_body(c):
            t, _, bm, bj = c
            bs = jnp.max(bm)
            alive = bs > _NEG / 2.0
            gi = jnp.where(bm == bs, bj * 16 + iota16, _BIGI)
            pos = jnp.min(gi)

            def do_pick():
                b0 = cx1[pl.ds(pos, 16)][0]
                b1 = cy1[pl.ds(pos, 16)][0]
                b2 = cx2[pl.ds(pos, 16)][0]
                b3 = cy2[pl.ds(pos, 16)][0]
                ba = car[pl.ds(pos, 16)][0]

                def su(j, carry):
                    nbm, nbj = carry
                    v = cs[pl.ds(j * 16, 16)]
                    vx1 = cx1[pl.ds(j * 16, 16)]
                    vy1 = cy1[pl.ds(j * 16, 16)]
                    vx2 = cx2[pl.ds(j * 16, 16)]
                    vy2 = cy2[pl.ds(j * 16, 16)]
                    va = car[pl.ds(j * 16, 16)]
                    ix1 = jnp.maximum(b0, vx1)
                    iy1 = jnp.maximum(b1, vy1)
                    ix2 = jnp.minimum(b2, vx2)
                    iy2 = jnp.minimum(b3, vy2)
                    inter = (jnp.maximum(ix2 - ix1, 0.0)
                             * jnp.maximum(iy2 - iy1, 0.0))
                    iou = inter / (va + ba - inter + 1e-8)
                    supp = (iou >= _IOU_THRESH) | (j * 16 + iota16 == pos)
                    newv = jnp.where(supp, _NEG, v)
                    cs[pl.ds(j * 16, 16)] = newv
                    better = newv > nbm
                    return (jnp.where(better, newv, nbm),
                            jnp.where(better, j, nbj))
                if unroll:
                    nbm, nbj = negv, zeros16i
                    for j in range(nv):
                        nbm, nbj = su(j, (nbm, nbj))
                else:
                    nbm, nbj = lax.fori_loop(0, nv, su, (negv, zeros16i))
                colv = jnp.full((16,), t, jnp.int32)
                lane0 = iota16 == 0
                for r, val in enumerate((b0, b1, b2, b3, bs,
                                         jnp.float32(1.0))):
                    plsc.store_scatter(
                        picks, [jnp.full((16,), r, jnp.int32), colv],
                        jnp.full((16,), val, jnp.float32), mask=lane0)
                return nbm, nbj

            def no_pick():
                return bm, bj

            nbm, nbj = lax.cond(alive, do_pick, no_pick)
            return t + alive.astype(jnp.int32), alive, nbm, nbj

        t, _, _, _ = lax.while_loop(w_cond, w_body, (0, True, bm0, bj0))
        return t

    def problem(k, _):
        q = sid * 5 + k
        p = cid * 80 + q
        img = p // _NUM_CLASSES
        base_idx = img * _N_PAD
        pltpu.sync_copy(sc_ref.at[p], srow)
        for r in range(6):
            for q in range(8):
                picks[r, pl.ds(q * 16, 16)] = jnp.zeros((16,), jnp.float32)

        c03 = count_above(jnp.float32(_SCORE_THRESH))

        def solve_dense():
            prefill(_NV2, base_idx)
            compact(jnp.float32(_SCORE_THRESH), base_idx)
            gather_boxes(_N_PAD // 128)
            greedy(_NV2)

        def solve_small():
            prefill(_NV1, base_idx)
            compact(jnp.float32(_SCORE_THRESH), base_idx)
            gather_boxes(_KCAP1 // 128)
            greedy(_NV1, unroll=True)

        def solve_truncated():
            smax = jnp.float32(16.0)

            def b_cond(c):
                lo, hi, tau, cnt, it = c
                return ((cnt < _MAX_PER_CLASS) | (cnt > _KCAP1)) & (it < 24)

            def b_body(c):
                lo, hi, tau, cnt, it = c
                mid = 0.5 * (lo + hi)
                first = jnp.float32(2.26)
                mid = jnp.where((it == 0) & (first > lo) & (first < hi),
                                first, mid)
                cc = count_above(mid)
                lo = jnp.where(cc > _KCAP1, mid, lo)
                hi = jnp.where(cc > _KCAP1, hi, mid)
                return lo, hi, mid, cc, it + 1

            lo0 = jnp.float32(_SCORE_THRESH)
            _, _, tau, cnt, _ = lax.while_loop(
                b_cond, b_body, (lo0, smax, lo0, c03, 0))

            def trunc_path():
                prefill(_NV1, base_idx)
                compact(tau, base_idx)
                gather_boxes(_KCAP1 // 128)
                npicks = greedy(_NV1, unroll=True)
                pl.when(npicks < _MAX_PER_CLASS)(solve_dense)

            lax.cond(cnt > _KCAP1, solve_dense, trunc_path)

        lax.cond(c03 <= _KCAP1, solve_small, solve_truncated)
        for r in range(6):
            pltpu.sync_copy(picks.at[r], shared.at[r, p])
        return 0

    lax.fori_loop(0, _NPROB // 32, problem, 0)
    plsc.subcore_barrier()

    @pl.when(sid < 4)
    def _merge():
        img = cid * 4 + sid
        copies = []
        for src_r, dst in ((0, cx1), (1, cy1), (2, cx2), (3, cy2),
                           (4, cs), (5, srow)):
            for j in range(_NUM_CLASSES):
                copies.append(pltpu.async_copy(
                    shared.at[src_r, img * _NUM_CLASSES + j],
                    dst.at[pl.ds(j * 128, 16 * 8)], sem))
        for h in copies:
            h.wait()
        def mk(j, _):
            sv = srow[pl.ds(j * 16, 16)]
            scv = cs[pl.ds(j * 16, 16)]
            cs[pl.ds(j * 16, 16)] = jnp.where(sv > 0.5, scv, _NEG)
            return 0
        lax.fori_loop(0, _NUM_CLASSES * 8, mk, 0)
        for j2 in range(64):
            obuf[pl.ds(j2 * 16, 16)] = jnp.zeros((16,), jnp.float32)

        h0 = plsc.load_gather(cs, [iota16 * 128])
        h1r = plsc.load_gather(cs, [(iota16 + 16) * 128])
        h1 = jnp.where(iota16 < _NUM_CLASSES - 16, h1r, _NEG)
        zi = jnp.zeros((16,), jnp.int32)

        def mstep(t2, carry):
            h0, h1, cu0, cu1, nv = carry
            bs = jnp.max(jnp.maximum(h0, h1))
            valid = bs > _NEG / 2.0
            cand = jnp.minimum(jnp.where(h0 == bs, iota16, _BIGI),
                               jnp.where(h1 == bs, iota16 + 16, _BIGI))
            cstar = jnp.min(cand)
            tcv = jnp.minimum(jnp.where(iota16 == cstar, cu0, _BIGI),
                              jnp.where(iota16 + 16 == cstar, cu1, _BIGI))
            tc = jnp.min(tcv)
            pos = cstar * 128 + tc

            @pl.when(valid)
            def _():
                b0 = jnp.clip(cx1[pl.ds(pos, 16)][0], 0.0, 1.0)
                b1 = jnp.clip(cy1[pl.ds(pos, 16)][0], 0.0, 1.0)
                b2 = jnp.clip(cx2[pl.ds(pos, 16)][0], 0.0, 1.0)
                b3 = jnp.clip(cy2[pl.ds(pos, 16)][0], 0.0, 1.0)
                clsf = cstar.astype(jnp.float32)
                x = jnp.where(iota16 == 0, b0,
                    jnp.where(iota16 == 1, b1,
                    jnp.where(iota16 == 2, b2,
                    jnp.where(iota16 == 3, b3,
                    jnp.where(iota16 == 4, bs, clsf)))))
                plsc.store_scatter(obuf, [iota16 * 128 + t2], x,
                                   mask=iota16 < 6)

            hv = cs[pl.ds(pos + 1, 16)][0]
            adv0 = valid & (iota16 == cstar)
            adv1 = valid & (iota16 + 16 == cstar)
            h0n = jnp.where(adv0, hv, h0)
            h1n = jnp.where(adv1, hv, h1)
            cu0n = jnp.where(adv0, tc + 1, cu0)
            cu1n = jnp.where(adv1, tc + 1, cu1)
            return h0n, h1n, cu0n, cu1n, nv + valid.astype(jnp.int32)

        _, _, _, _, nv = lax.fori_loop(
            0, _MAX_TOTAL, mstep, (h0, h1, zi, zi, 0))
        nvf = jnp.full((16,), nv.astype(jnp.float32))
        for j3 in range(8):
            obuf[pl.ds(6 * 128 + j3 * 16, 16)] = nvf
        pltpu.sync_copy(obuf, out_ref.at[img])


def _merge_kernel(x1_ref, y1_ref, x2_ref, y2_ref, sc_ref, vd_ref, out_ref):
    B, W = _BATCH, _NUM_CLASSES * 128
    sx1, sy1 = x1_ref[...], y1_ref[...]
    sx2, sy2 = x2_ref[...], y2_ref[...]
    ssc, svd = sc_ref[...], vd_ref[...]

    key = jnp.where(svd > 0.5, ssc, _NEG)
    fmat = lax.broadcasted_iota(jnp.int32, (B, W), 1)
    out_iota = lax.broadcasted_iota(jnp.int32, (B, 128), 1)
    zo = jnp.zeros((B, 128), jnp.float32)

    def merge_step(t2, carry):
        key, ob0, ob1, ob2, ob3, osc, ocl, nv = carry
        mv = jnp.max(key, axis=1, keepdims=True)
        eq2 = key == mv
        bf = jnp.min(jnp.where(eq2, fmat, _BIGI), axis=1,
                     keepdims=True)
        oh2 = fmat == bf
        ninf = jnp.float32(-3e38)
        g0 = jnp.max(jnp.where(oh2, sx1, ninf), axis=1, keepdims=True)
        g1 = jnp.max(jnp.where(oh2, sy1, ninf), axis=1, keepdims=True)
        g2 = jnp.max(jnp.where(oh2, sx2, ninf), axis=1, keepdims=True)
        g3 = jnp.max(jnp.where(oh2, sy2, ninf), axis=1, keepdims=True)
        vflag = mv > _NEG / 2.0
        clsf = (bf // 128).astype(jnp.float32)
        vz = jnp.float32(0.0)
        c0 = jnp.where(vflag, jnp.clip(g0, 0.0, 1.0), vz)
        c1 = jnp.where(vflag, jnp.clip(g1, 0.0, 1.0), vz)
        c2 = jnp.where(vflag, jnp.clip(g2, 0.0, 1.0), vz)
        c3 = jnp.where(vflag, jnp.clip(g3, 0.0, 1.0), vz)
        cs = jnp.where(vflag, mv, vz)
        cc = jnp.where(vflag, clsf, vz)
        sel2 = out_iota == t2
        ob0 = jnp.where(sel2, c0, ob0)
        ob1 = jnp.where(sel2, c1, ob1)
        ob2 = jnp.where(sel2, c2, ob2)
        ob3 = jnp.where(sel2, c3, ob3)
        osc = jnp.where(sel2, cs, osc)
        ocl = jnp.where(sel2, cc, ocl)
        nv = nv + jnp.where(vflag, 1.0, 0.0)
        key = jnp.where(oh2, jnp.float32(-3e38), key)
        return key, ob0, ob1, ob2, ob3, osc, ocl, nv

    _, ob0, ob1, ob2, ob3, osc, ocl, nv = lax.fori_loop(
        0, _MAX_TOTAL, merge_step,
        (key, zo, zo, zo, zo, zo, zo, jnp.zeros((B, 1), jnp.float32)))

    out_ref[:, 0:128] = ob0
    out_ref[:, 128:256] = ob1
    out_ref[:, 256:384] = ob2
    out_ref[:, 384:512] = ob3
    out_ref[:, 512:640] = osc
    out_ref[:, 640:768] = ocl
    out_ref[:, 768:896] = jnp.broadcast_to(nv, (B, 128))
    out_ref[:, 896:1024] = zo


@jax.jit
def _run(predictions, anchors_t):
    B, N, C = _BATCH, _N_ANCH, _NUM_CLASSES
    padn = _N_PAD - N
    box_pred = jnp.transpose(predictions[:, :, :4], (0, 2, 1))
    box_pred = jnp.pad(box_pred, ((0, 0), (0, 0), (0, padn)))
    scores = jnp.transpose(predictions[:, :, 4:], (0, 2, 1))
    scores = jnp.pad(scores, ((0, 0), (0, 0), (0, padn)))
    scores = scores.reshape(_NPROB, _N_PAD)

    table = pl.pallas_call(
        _decode_kernel,
        grid=(B,),
        in_specs=[
            pl.BlockSpec((1, 4, _N_PAD), lambda i: (i, 0, 0)),
            pl.BlockSpec((4, _N_PAD), lambda i: (0, 0)),
        ],
        out_specs=pl.BlockSpec((1, 5, _N_PAD), lambda i: (i, 0, 0)),
        out_shape=jax.ShapeDtypeStruct((B, 5, _N_PAD), jnp.float32),
    )(box_pred, anchors_t)
    tflat = jnp.transpose(table, (1, 0, 2)).reshape(5, B * _N_PAD)

    mesh = plsc.VectorSubcoreMesh(core_axis_name="c", subcore_axis_name="s",
                                  num_cores=2, num_subcores=16)
    _, out = pl.kernel(
        _sc_nms,
        out_type=(jax.ShapeDtypeStruct((6, _NPROB, 128), jnp.float32),
                  jax.ShapeDtypeStruct((B, 1024), jnp.float32)),
        mesh=mesh,
        compiler_params=pltpu.CompilerParams(needs_layout_passes=False),
        scratch_types=[
            pltpu.VMEM((_N_PAD,), jnp.float32),
            pltpu.VMEM((_CBUF,), jnp.float32),
            pltpu.VMEM((_CBUF,), jnp.int32),
            pltpu.VMEM((_CBUF,), jnp.float32),
            pltpu.VMEM((_CBUF,), jnp.float32),
            pltpu.VMEM((_CBUF,), jnp.float32),
            pltpu.VMEM((_CBUF,), jnp.float32),
            pltpu.VMEM((_CBUF,), jnp.float32),
            pltpu.VMEM((6, 128), jnp.float32),
            pltpu.VMEM((1024,), jnp.float32),
            pltpu.SemaphoreType.DMA,
        ],
    )(scores, tflat[0], tflat[1], tflat[2], tflat[3], tflat[4])

    M = _MAX_TOTAL
    out3 = out.reshape(B, 8, 128)
    boxes = jnp.transpose(out3[:, 0:4, :M], (0, 2, 1))
    nmsed_scores = out3[:, 4, :M]
    nmsed_classes = out3[:, 5, :M]
    n_valid = out3[:, 6, 0].astype(jnp.int32)
    return boxes, nmsed_scores, nmsed_classes, n_valid


_ANCHORS_T = _anchors_t()


def kernel(predictions):
    return _run(predictions, _ANCHORS_T)
